# Initial kernel scaffold; baseline (speedup 1.0000x reference)
import jax
import jax.numpy as jnp
from jax.experimental import pallas as pl


def kernel(x, edge_index, edge_attr, batch, atom_tables, bond_tables, edge_W, edge_b, pre_W, pre_b, post_W, post_b, lin_W, lin_b, bn_w, bn_b, bn_rm, bn_rv, proj1_W, proj1_b, proj2_W, proj2_b):
    raise NotImplementedError("write your pallas kernel here")


# speedup vs baseline: 10.8293x; 10.8293x over previous
import jax
import jax.numpy as jnp
from jax.experimental import pallas as pl

_N = 10000
_E = 160000
_H = 128
_T = 4
_FO = 32
_L = 4
_G = 256
_OUT = 768
_AVG_LOG = 2.833213344056216
_TH = _T * _H
_BE = 1600
_BN = 1000


def _edge_kernel(e_ref, hd_ref, hs_ref, eW_ref, eb_ref, W2_ref, pb_ref, out_ref):
    ee = jnp.dot(e_ref[...], eW_ref[...], preferred_element_type=jnp.float32) + eb_ref[...]
    W2 = W2_ref[...]
    acc = jnp.dot(hd_ref[...], W2[:_H], preferred_element_type=jnp.float32)
    acc = acc + jnp.dot(hs_ref[...], W2[_H:2 * _H], preferred_element_type=jnp.float32)
    acc = acc + jnp.dot(ee, W2[2 * _H:], preferred_element_type=jnp.float32)
    out_ref[...] = acc + pb_ref[...]


def _node_kernel(h_ref, s1_ref, s2_ref, mn_ref, mx_ref, cnt_ref,
                 Wx_ref, Wa_ref, Wb_ref, Wc_ref, pb_ref,
                 linW_ref, linb_ref, sc_ref, bi_ref, out_ref):
    h = h_ref[...]
    cnt = cnt_ref[...]
    cnt_c = jnp.maximum(cnt, 1.0)
    lg = jnp.log(cnt_c + 1.0)
    amp = lg / _AVG_LOG
    att = _AVG_LOG / lg
    has = cnt > 0.0
    otx = jnp.dot(h, Wx_ref[...], preferred_element_type=jnp.float32)
    parts = []
    for t in range(_T):
        sl = slice(t * _H, (t + 1) * _H)
        s1t = s1_ref[:, sl]
        s2t = s2_ref[:, sl]
        mnt = jnp.where(has, mn_ref[:, sl], 0.0)
        mxt = jnp.where(has, mx_ref[:, sl], 0.0)
        mean = s1t / cnt_c
        var = s2t / cnt_c - mean * mean
        std = jnp.sqrt(jnp.maximum(var, 0.0) + 1e-5)
        aggr = jnp.concatenate([mean, mnt, mxt, std], axis=1)
        ra = jnp.dot(aggr, Wa_ref[t], preferred_element_type=jnp.float32)
        rb = jnp.dot(aggr, Wb_ref[t], preferred_element_type=jnp.float32)
        rc = jnp.dot(aggr, Wc_ref[t], preferred_element_type=jnp.float32)
        fs = slice(t * _FO, (t + 1) * _FO)
        parts.append(otx[:, fs] + ra + amp * rb + att * rc + pb_ref[:, fs])
    ot = jnp.concatenate(parts, axis=1)
    y = jnp.dot(ot, linW_ref[...], preferred_element_type=jnp.float32) + linb_ref[...]
    y = y * sc_ref[...] + bi_ref[...]
    y = jnp.maximum(y, 0.0)
    out_ref[...] = h + y


def _proj_kernel(g_ref, p1_ref, b1_ref, p2_ref, b2_ref, out_ref):
    g = jnp.maximum(jnp.dot(g_ref[...], p1_ref[...], preferred_element_type=jnp.float32) + b1_ref[...], 0.0)
    g = jnp.dot(g, p2_ref[...], preferred_element_type=jnp.float32) + b2_ref[...]
    nrm = jnp.sqrt(jnp.sum(g * g, axis=1, keepdims=True))
    out_ref[...] = g / jnp.maximum(nrm, 1e-12)


def kernel(x, edge_index, edge_attr, batch, atom_tables, bond_tables, edge_W, edge_b, pre_W, pre_b, post_W, post_b, lin_W, lin_b, bn_w, bn_b, bn_rm, bn_rv, proj1_W, proj1_b, proj2_W, proj2_b):
    f32 = jnp.float32
    h = jnp.zeros((_N, _H), f32)
    for i in range(9):
        h = h + jnp.take(atom_tables[i], x[:, i], axis=0)
    e = jnp.zeros((_E, _H), f32)
    for i in range(3):
        e = e + jnp.take(bond_tables[i], edge_attr[:, i], axis=0)
    src = edge_index[0]
    dst = edge_index[1]
    cnt = jnp.zeros((_N,), f32).at[dst].add(1.0)
    cnt2 = cnt.reshape(_N, 1)

    edge_grid = pl.pallas_call(
        _edge_kernel,
        grid=(_E // _BE,),
        in_specs=[
            pl.BlockSpec((_BE, _H), lambda i: (i, 0)),
            pl.BlockSpec((_BE, _H), lambda i: (i, 0)),
            pl.BlockSpec((_BE, _H), lambda i: (i, 0)),
            pl.BlockSpec((_H, _H), lambda i: (0, 0)),
            pl.BlockSpec((1, _H), lambda i: (0, 0)),
            pl.BlockSpec((3 * _H, _TH), lambda i: (0, 0)),
            pl.BlockSpec((1, _TH), lambda i: (0, 0)),
        ],
        out_specs=pl.BlockSpec((_BE, _TH), lambda i: (i, 0)),
        out_shape=jax.ShapeDtypeStruct((_E, _TH), f32),
    )

    node_grid = pl.pallas_call(
        _node_kernel,
        grid=(_N // _BN,),
        in_specs=[
            pl.BlockSpec((_BN, _H), lambda i: (i, 0)),
            pl.BlockSpec((_BN, _TH), lambda i: (i, 0)),
            pl.BlockSpec((_BN, _TH), lambda i: (i, 0)),
            pl.BlockSpec((_BN, _TH), lambda i: (i, 0)),
            pl.BlockSpec((_BN, _TH), lambda i: (i, 0)),
            pl.BlockSpec((_BN, 1), lambda i: (i, 0)),
            pl.BlockSpec((_H, _T * _FO), lambda i: (0, 0)),
            pl.BlockSpec((_T, 4 * _H, _FO), lambda i: (0, 0, 0)),
            pl.BlockSpec((_T, 4 * _H, _FO), lambda i: (0, 0, 0)),
            pl.BlockSpec((_T, 4 * _H, _FO), lambda i: (0, 0, 0)),
            pl.BlockSpec((1, _T * _FO), lambda i: (0, 0)),
            pl.BlockSpec((_T * _FO, _H), lambda i: (0, 0)),
            pl.BlockSpec((1, _H), lambda i: (0, 0)),
            pl.BlockSpec((1, _H), lambda i: (0, 0)),
            pl.BlockSpec((1, _H), lambda i: (0, 0)),
        ],
        out_specs=pl.BlockSpec((_BN, _H), lambda i: (i, 0)),
        out_shape=jax.ShapeDtypeStruct((_N, _H), f32),
    )

    for l in range(_L):
        W2 = pre_W[l].transpose(1, 0, 2).reshape(3 * _H, _TH)
        pb2 = pre_b[l].reshape(1, _TH)
        Wx = post_W[l][:, :_H, :].transpose(1, 0, 2).reshape(_H, _T * _FO)
        Wa = post_W[l][:, _H:5 * _H, :]
        Wb = post_W[l][:, 5 * _H:9 * _H, :]
        Wc = post_W[l][:, 9 * _H:13 * _H, :]
        pb = post_b[l].reshape(1, _T * _FO)
        sc = (bn_w[l] / jnp.sqrt(bn_rv[l] + 1e-5)).reshape(1, _H)
        bi = (bn_b[l] - bn_rm[l] * sc[0]).reshape(1, _H)

        hd = jnp.take(h, dst, axis=0)
        hsrc = jnp.take(h, src, axis=0)
        hs = edge_grid(e, hd, hsrc, edge_W[l], edge_b[l].reshape(1, _H), W2, pb2)
        s1 = jax.ops.segment_sum(hs, dst, num_segments=_N)
        s2 = jax.ops.segment_sum(hs * hs, dst, num_segments=_N)
        mn = jax.ops.segment_min(hs, dst, num_segments=_N)
        mx = jax.ops.segment_max(hs, dst, num_segments=_N)
        h = node_grid(h, s1, s2, mn, mx, cnt2, Wx, Wa, Wb, Wc, pb,
                      lin_W[l], lin_b[l].reshape(1, _H), sc, bi)

    g = jnp.zeros((_G, _H), f32).at[batch].add(h)
    out = pl.pallas_call(
        _proj_kernel,
        grid=(1,),
        in_specs=[
            pl.BlockSpec((_G, _H), lambda i: (0, 0)),
            pl.BlockSpec((_H, _H), lambda i: (0, 0)),
            pl.BlockSpec((1, _H), lambda i: (0, 0)),
            pl.BlockSpec((_H, _OUT), lambda i: (0, 0)),
            pl.BlockSpec((1, _OUT), lambda i: (0, 0)),
        ],
        out_specs=pl.BlockSpec((_G, _OUT), lambda i: (0, 0)),
        out_shape=jax.ShapeDtypeStruct((_G, _OUT), f32),
    )(g, proj1_W, proj1_b.reshape(1, _H), proj2_W, proj2_b.reshape(1, _OUT))
    return out


# speedup vs baseline: 11.1843x; 1.0328x over previous
import jax
import jax.numpy as jnp
from jax.experimental import pallas as pl

_N = 10000
_E = 160000
_H = 128
_T = 4
_FO = 32
_L = 4
_G = 256
_OUT = 768
_AVG_LOG = 2.833213344056216
_TH = _T * _H
_BE = 1600
_BN = 1000


def _edge_kernel(e_ref, hd_ref, hs_ref, eW_ref, eb_ref, W2_ref, pb_ref, out_ref):
    ee = jnp.dot(e_ref[...], eW_ref[...], preferred_element_type=jnp.float32) + eb_ref[...]
    W2 = W2_ref[...]
    acc = jnp.dot(hd_ref[...], W2[:_H], preferred_element_type=jnp.float32)
    acc = acc + jnp.dot(hs_ref[...], W2[_H:2 * _H], preferred_element_type=jnp.float32)
    acc = acc + jnp.dot(ee, W2[2 * _H:], preferred_element_type=jnp.float32)
    out_ref[...] = acc + pb_ref[...]


def _node_kernel(h_ref, s1_ref, s2_ref, mn_ref, mx_ref, cnt_ref,
                 Wx_ref, Wa_ref, Wb_ref, Wc_ref, pb_ref,
                 linW_ref, linb_ref, sc_ref, bi_ref, out_ref):
    h = h_ref[...]
    cnt = cnt_ref[...]
    cnt_c = jnp.maximum(cnt, 1.0)
    lg = jnp.log(cnt_c + 1.0)
    amp = lg / _AVG_LOG
    att = _AVG_LOG / lg
    has = cnt > 0.0
    otx = jnp.dot(h, Wx_ref[...], preferred_element_type=jnp.float32)
    parts = []
    for t in range(_T):
        sl = slice(t * _H, (t + 1) * _H)
        s1t = s1_ref[:, sl]
        s2t = s2_ref[:, sl]
        mnt = jnp.where(has, mn_ref[:, sl], 0.0)
        mxt = jnp.where(has, mx_ref[:, sl], 0.0)
        mean = s1t / cnt_c
        var = s2t / cnt_c - mean * mean
        std = jnp.sqrt(jnp.maximum(var, 0.0) + 1e-5)
        aggr = jnp.concatenate([mean, mnt, mxt, std], axis=1)
        ra = jnp.dot(aggr, Wa_ref[t], preferred_element_type=jnp.float32)
        rb = jnp.dot(aggr, Wb_ref[t], preferred_element_type=jnp.float32)
        rc = jnp.dot(aggr, Wc_ref[t], preferred_element_type=jnp.float32)
        fs = slice(t * _FO, (t + 1) * _FO)
        parts.append(otx[:, fs] + ra + amp * rb + att * rc + pb_ref[:, fs])
    ot = jnp.concatenate(parts, axis=1)
    y = jnp.dot(ot, linW_ref[...], preferred_element_type=jnp.float32) + linb_ref[...]
    y = y * sc_ref[...] + bi_ref[...]
    y = jnp.maximum(y, 0.0)
    out_ref[...] = h + y


def _proj_kernel(g_ref, p1_ref, b1_ref, p2_ref, b2_ref, out_ref):
    g = jnp.maximum(jnp.dot(g_ref[...], p1_ref[...], preferred_element_type=jnp.float32) + b1_ref[...], 0.0)
    g = jnp.dot(g, p2_ref[...], preferred_element_type=jnp.float32) + b2_ref[...]
    nrm = jnp.sqrt(jnp.sum(g * g, axis=1, keepdims=True))
    out_ref[...] = g / jnp.maximum(nrm, 1e-12)


def kernel(x, edge_index, edge_attr, batch, atom_tables, bond_tables, edge_W, edge_b, pre_W, pre_b, post_W, post_b, lin_W, lin_b, bn_w, bn_b, bn_rm, bn_rv, proj1_W, proj1_b, proj2_W, proj2_b):
    f32 = jnp.float32
    h = jnp.zeros((_N, _H), f32)
    for i in range(9):
        h = h + jnp.take(atom_tables[i], x[:, i], axis=0)
    e = jnp.zeros((_E, _H), f32)
    for i in range(3):
        e = e + jnp.take(bond_tables[i], edge_attr[:, i], axis=0)
    order = jnp.argsort(edge_index[1])
    src = edge_index[0][order]
    dst = edge_index[1][order]
    e = jnp.take(e, order, axis=0)
    cnt = jnp.zeros((_N,), f32).at[dst].add(1.0)
    cnt2 = cnt.reshape(_N, 1)

    edge_grid = pl.pallas_call(
        _edge_kernel,
        grid=(_E // _BE,),
        in_specs=[
            pl.BlockSpec((_BE, _H), lambda i: (i, 0)),
            pl.BlockSpec((_BE, _H), lambda i: (i, 0)),
            pl.BlockSpec((_BE, _H), lambda i: (i, 0)),
            pl.BlockSpec((_H, _H), lambda i: (0, 0)),
            pl.BlockSpec((1, _H), lambda i: (0, 0)),
            pl.BlockSpec((3 * _H, _TH), lambda i: (0, 0)),
            pl.BlockSpec((1, _TH), lambda i: (0, 0)),
        ],
        out_specs=pl.BlockSpec((_BE, _TH), lambda i: (i, 0)),
        out_shape=jax.ShapeDtypeStruct((_E, _TH), f32),
    )

    node_grid = pl.pallas_call(
        _node_kernel,
        grid=(_N // _BN,),
        in_specs=[
            pl.BlockSpec((_BN, _H), lambda i: (i, 0)),
            pl.BlockSpec((_BN, _TH), lambda i: (i, 0)),
            pl.BlockSpec((_BN, _TH), lambda i: (i, 0)),
            pl.BlockSpec((_BN, _TH), lambda i: (i, 0)),
            pl.BlockSpec((_BN, _TH), lambda i: (i, 0)),
            pl.BlockSpec((_BN, 1), lambda i: (i, 0)),
            pl.BlockSpec((_H, _T * _FO), lambda i: (0, 0)),
            pl.BlockSpec((_T, 4 * _H, _FO), lambda i: (0, 0, 0)),
            pl.BlockSpec((_T, 4 * _H, _FO), lambda i: (0, 0, 0)),
            pl.BlockSpec((_T, 4 * _H, _FO), lambda i: (0, 0, 0)),
            pl.BlockSpec((1, _T * _FO), lambda i: (0, 0)),
            pl.BlockSpec((_T * _FO, _H), lambda i: (0, 0)),
            pl.BlockSpec((1, _H), lambda i: (0, 0)),
            pl.BlockSpec((1, _H), lambda i: (0, 0)),
            pl.BlockSpec((1, _H), lambda i: (0, 0)),
        ],
        out_specs=pl.BlockSpec((_BN, _H), lambda i: (i, 0)),
        out_shape=jax.ShapeDtypeStruct((_N, _H), f32),
    )

    for l in range(_L):
        W2 = pre_W[l].transpose(1, 0, 2).reshape(3 * _H, _TH)
        pb2 = pre_b[l].reshape(1, _TH)
        Wx = post_W[l][:, :_H, :].transpose(1, 0, 2).reshape(_H, _T * _FO)
        Wa = post_W[l][:, _H:5 * _H, :]
        Wb = post_W[l][:, 5 * _H:9 * _H, :]
        Wc = post_W[l][:, 9 * _H:13 * _H, :]
        pb = post_b[l].reshape(1, _T * _FO)
        sc = (bn_w[l] / jnp.sqrt(bn_rv[l] + 1e-5)).reshape(1, _H)
        bi = (bn_b[l] - bn_rm[l] * sc[0]).reshape(1, _H)

        hd = jnp.take(h, dst, axis=0)
        hsrc = jnp.take(h, src, axis=0)
        hs = edge_grid(e, hd, hsrc, edge_W[l], edge_b[l].reshape(1, _H), W2, pb2)
        s1 = jax.ops.segment_sum(hs, dst, num_segments=_N, indices_are_sorted=True)
        s2 = jax.ops.segment_sum(hs * hs, dst, num_segments=_N, indices_are_sorted=True)
        mn = jax.ops.segment_min(hs, dst, num_segments=_N, indices_are_sorted=True)
        mx = jax.ops.segment_max(hs, dst, num_segments=_N, indices_are_sorted=True)
        h = node_grid(h, s1, s2, mn, mx, cnt2, Wx, Wa, Wb, Wc, pb,
                      lin_W[l], lin_b[l].reshape(1, _H), sc, bi)

    g = jnp.zeros((_G, _H), f32).at[batch].add(h)
    out = pl.pallas_call(
        _proj_kernel,
        grid=(1,),
        in_specs=[
            pl.BlockSpec((_G, _H), lambda i: (0, 0)),
            pl.BlockSpec((_H, _H), lambda i: (0, 0)),
            pl.BlockSpec((1, _H), lambda i: (0, 0)),
            pl.BlockSpec((_H, _OUT), lambda i: (0, 0)),
            pl.BlockSpec((1, _OUT), lambda i: (0, 0)),
        ],
        out_specs=pl.BlockSpec((_G, _OUT), lambda i: (0, 0)),
        out_shape=jax.ShapeDtypeStruct((_G, _OUT), f32),
    )(g, proj1_W, proj1_b.reshape(1, _H), proj2_W, proj2_b.reshape(1, _OUT))
    return out
